# Initial kernel scaffold; baseline (speedup 1.0000x reference)
import jax
import jax.numpy as jnp
from jax.experimental import pallas as pl


def kernel(x, W_enc, b_enc, D, b_dec):
    raise NotImplementedError("write your pallas kernel here")


# speedup vs baseline: 72.4325x; 72.4325x over previous
import jax
import jax.numpy as jnp
from jax.experimental import pallas as pl
from jax.experimental.pallas import tpu as pltpu

_ACT = 768
_DICT = 16384
_K = 256
_NTOK = 4096

_TT = 512
_TD = 2048
_CH = 512
_NCHUNK = _NTOK * _DICT // _CH

_TDB = 2048
_TT2 = 512


def _enc_kernel(x_ref, w_ref, benc_ref, bdec_ref, pre_ref, mx_ref):
    xa = x_ref[...] - bdec_ref[...]
    acts = jax.lax.dot_general(
        xa, w_ref[...], (((1,), (1,)), ((), ())),
        preferred_element_type=jnp.float32)
    acts = jnp.maximum(acts + benc_ref[...], 0.0)
    pre_ref[...] = acts
    mx = jnp.max(acts.reshape(_TT, _TD // _CH, _CH), axis=2)
    mx_ref[...] = mx.reshape(1, _TT, _TD // _CH)


def _gather_kernel(ids_ref, chunk_ref, out_ref):
    out_ref[...] = chunk_ref[...]


def _dg_kernel(dicts_ref, vals_ref, d_ref, out_ref):
    j = pl.program_id(0)
    col = jax.lax.broadcasted_iota(jnp.int32, (_K, _TDB), 1) + j * _TDB
    onehot = jnp.where(dicts_ref[...] == col, vals_ref[...], 0.0)

    @pl.when(j == 0)
    def _():
        out_ref[...] = jnp.zeros_like(out_ref)

    out_ref[...] += jnp.dot(onehot, d_ref[...],
                            preferred_element_type=jnp.float32)


def _dec_kernel(tokens_ref, dg_ref, bdec_ref, out_ref):
    i = pl.program_id(0)
    rows = jax.lax.broadcasted_iota(jnp.int32, (_TT2, _K), 0) + i * _TT2
    m = jnp.where(rows == tokens_ref[...], 1.0, 0.0)
    out_ref[...] = jnp.dot(m, dg_ref[...],
                           preferred_element_type=jnp.float32) + bdec_ref[...]


def kernel(x, W_enc, b_enc, D, b_dec):
    benc2 = b_enc.reshape(1, _DICT)
    bdec2 = b_dec.reshape(1, _ACT)

    pre, mx = pl.pallas_call(
        _enc_kernel,
        grid=(_NTOK // _TT, _DICT // _TD),
        in_specs=[
            pl.BlockSpec((_TT, _ACT), lambda i, j: (i, 0)),
            pl.BlockSpec((_TD, _ACT), lambda i, j: (j, 0)),
            pl.BlockSpec((1, _TD), lambda i, j: (0, j)),
            pl.BlockSpec((1, _ACT), lambda i, j: (0, 0)),
        ],
        out_specs=[
            pl.BlockSpec((_TT, _TD), lambda i, j: (i, j)),
            pl.BlockSpec((1, _TT, _TD // _CH), lambda i, j: (j, i, 0)),
        ],
        out_shape=[
            jax.ShapeDtypeStruct((_NTOK, _DICT), jnp.float32),
            jax.ShapeDtypeStruct(
                (_DICT // _TD, _NTOK, _TD // _CH), jnp.float32),
        ],
        compiler_params=pltpu.CompilerParams(
            dimension_semantics=("parallel", "parallel")),
    )(x, W_enc, benc2, bdec2)

    mx_flat = jnp.transpose(mx, (1, 0, 2)).reshape(-1)
    _, chunk_ids = jax.lax.top_k(mx_flat, _K)

    pre_chunks = pre.reshape(_NCHUNK, 1, _CH)
    cand = pl.pallas_call(
        _gather_kernel,
        grid_spec=pltpu.PrefetchScalarGridSpec(
            num_scalar_prefetch=1,
            grid=(_K,),
            in_specs=[pl.BlockSpec((1, 1, _CH), lambda i, ids: (ids[i], 0, 0))],
            out_specs=pl.BlockSpec((1, 1, _CH), lambda i, ids: (i, 0, 0)),
        ),
        out_shape=jax.ShapeDtypeStruct((_K, 1, _CH), jnp.float32),
    )(chunk_ids, pre_chunks)

    vals, pos = jax.lax.top_k(cand.reshape(-1), _K)
    flat_idx = chunk_ids[pos // _CH] * _CH + pos % _CH
    token = flat_idx // _DICT
    dct = flat_idx % _DICT

    dg = pl.pallas_call(
        _dg_kernel,
        grid=(_DICT // _TDB,),
        in_specs=[
            pl.BlockSpec((_K, 1), lambda j: (0, 0)),
            pl.BlockSpec((_K, 1), lambda j: (0, 0)),
            pl.BlockSpec((_TDB, _ACT), lambda j: (j, 0)),
        ],
        out_specs=pl.BlockSpec((_K, _ACT), lambda j: (0, 0)),
        out_shape=jax.ShapeDtypeStruct((_K, _ACT), jnp.float32),
        compiler_params=pltpu.CompilerParams(
            dimension_semantics=("arbitrary",)),
    )(dct.reshape(_K, 1), vals.reshape(_K, 1), D)

    x_hat = pl.pallas_call(
        _dec_kernel,
        grid=(_NTOK // _TT2,),
        in_specs=[
            pl.BlockSpec((1, _K), lambda i: (0, 0)),
            pl.BlockSpec((_K, _ACT), lambda i: (0, 0)),
            pl.BlockSpec((1, _ACT), lambda i: (0, 0)),
        ],
        out_specs=pl.BlockSpec((_TT2, _ACT), lambda i: (i, 0)),
        out_shape=jax.ShapeDtypeStruct((_NTOK, _ACT), jnp.float32),
        compiler_params=pltpu.CompilerParams(
            dimension_semantics=("parallel",)),
    )(token.reshape(1, _K), dg, bdec2)

    return x_hat
